# Initial kernel scaffold; baseline (speedup 1.0000x reference)
import jax
import jax.numpy as jnp
from jax.experimental import pallas as pl


def kernel(x, edge_index, params):
    raise NotImplementedError("write your pallas kernel here")


# speedup vs baseline: 1.7364x; 1.7364x over previous
import functools
import numpy as np
import jax
import jax.numpy as jnp
from jax.experimental import pallas as pl

_INV = float(1.0 / np.sqrt(1.0 + 1e-5))


def _first_body(x_ref, w_ref, b_ref, o_ref):
    o_ref[...] = jnp.maximum(
        jnp.dot(x_ref[...], w_ref[...], preferred_element_type=jnp.float32)
        + b_ref[...], 0.0)


def _pq_body(h_ref, wp_ref, wq_ref, p_ref, q_ref):
    h = h_ref[...]
    p_ref[...] = jnp.dot(h, wp_ref[...], preferred_element_type=jnp.float32)
    q_ref[...] = jnp.dot(h, wq_ref[...], preferred_element_type=jnp.float32)


def _combine_body(p_ref, qm_ref, c_ref, o_ref):
    qm = qm_ref[...]
    agg = jnp.where(qm > 1e37, 0.0, p_ref[...] + c_ref[...] - qm)
    o_ref[...] = jnp.maximum(agg, 0.0)


def _dense(body, outs, *args):
    return pl.pallas_call(
        body,
        out_shape=outs,
    )(*args)


def kernel(x, edge_index, params):
    p = params
    src, dst = edge_index[0], edge_index[1]
    n = x.shape[0]

    w0 = p['first_W'] * (_INV * p['first_gamma'])[None, :]
    b0 = (p['first_b'] * _INV * p['first_gamma'] + p['first_beta'])[None, :]
    h = _dense(_first_body, jax.ShapeDtypeStruct((n, 128), jnp.float32),
               x, w0, b0)

    means = [jnp.mean(h, axis=0, keepdims=True)]
    for i in range(1, 4):
        s = _INV * p['conv%d_gamma' % i]
        wp = (p['conv%d_theta_W' % i] + p['conv%d_phi_W' % i]) * s[None, :]
        wq = p['conv%d_theta_W' % i] * s[None, :]
        c = ((p['conv%d_theta_b' % i] + p['conv%d_phi_b' % i]) * s
             + p['conv%d_beta' % i])[None, :]
        d = wp.shape[1]
        P, Q = _dense(
            _pq_body,
            (jax.ShapeDtypeStruct((n, d), jnp.float32),
             jax.ShapeDtypeStruct((n, d), jnp.float32)),
            h, wp, wq)
        qmin = jax.ops.segment_min(Q[src], dst, num_segments=n)
        h = _dense(_combine_body, jax.ShapeDtypeStruct((n, d), jnp.float32),
                   P, qmin, c)
        means.append(jnp.mean(h, axis=0, keepdims=True))

    out_feat = means[0] @ p['lin0_W'] + p['lin0_b']
    for i in range(1, 4):
        out_feat = out_feat + means[i] @ p['lin%d_W' % i] + p['lin%d_b' % i]
    out = out_feat @ p['cls_W'] + p['cls_b']
    return out, out_feat, h


# speedup vs baseline: 3.2063x; 1.8465x over previous
import functools
import numpy as np
import jax
import jax.numpy as jnp
from jax import lax
from jax.experimental import pallas as pl
from jax.experimental.pallas import tpu as pltpu, tpu_sc as plsc

_INV = float(1.0 / np.sqrt(1.0 + 1e-5))

NW = 32
KC = 4000
KB = 128


def _first_body(nvalid, x_ref, w_ref, b_ref, wl_ref, bl_ref, h_ref, z_ref):
    h = jnp.maximum(
        jnp.dot(x_ref[...], w_ref[...], preferred_element_type=jnp.float32)
        + b_ref[...], 0.0)
    rows = lax.broadcasted_iota(jnp.int32, h.shape, 0)
    h = jnp.where(rows < nvalid, h, 0.0)
    h_ref[...] = h
    cs = jnp.sum(h, axis=0, keepdims=True) * (1.0 / nvalid)
    z_ref[...] = jnp.dot(cs, wl_ref[...], preferred_element_type=jnp.float32) + bl_ref[...]


def _pq_body(h_ref, wp_ref, wq_ref, p_ref, q_ref):
    h = h_ref[...]
    p_ref[...] = jnp.dot(h, wp_ref[...], preferred_element_type=jnp.float32)
    q_ref[...] = jnp.dot(h, wq_ref[...], preferred_element_type=jnp.float32)


def _combine_body(nvalid, p_ref, qm_ref, c_ref, wl_ref, bl_ref, h_ref, z_ref):
    qm = qm_ref[...]
    agg = jnp.where(qm > 1e37, 0.0, p_ref[...] + c_ref[...] - qm)
    h = jnp.maximum(agg, 0.0)
    h_ref[...] = h
    cs = jnp.sum(h, axis=0, keepdims=True) * (1.0 / nvalid)
    z_ref[...] = jnp.dot(cs, wl_ref[...], preferred_element_type=jnp.float32) + bl_ref[...]


@functools.lru_cache(maxsize=None)
def _make_edge_prep(E, NB, CAP):
    mesh = plsc.VectorSubcoreMesh(core_axis_name="c", subcore_axis_name="s")
    n_chunks = E // KC
    assert E % KC == 0

    @functools.partial(
        pl.kernel,
        out_type=(jax.ShapeDtypeStruct((NW * CAP,), jnp.int32),
                  jax.ShapeDtypeStruct((NW * 16,), jnp.int32)),
        mesh=mesh,
        compiler_params=pltpu.CompilerParams(needs_layout_passes=False),
        scratch_types=[pltpu.VMEM((KC,), jnp.int32),
                       pltpu.VMEM((KC,), jnp.int32),
                       pltpu.VMEM((KC + 16,), jnp.int32),
                       pltpu.VMEM((16,), jnp.int32)],
    )
    def prep(src_hbm, dst_hbm, epk_hbm, cnt_hbm,
             srcv, dstv, stage, cstage):
        w = lax.axis_index("c") * 16 + lax.axis_index("s")
        lo = w * NB

        dummy = jnp.full((16,), NB, jnp.int32)

        def chunk_body(g, cursor):
            pltpu.sync_copy(src_hbm.at[pl.ds(g * KC, KC)], srcv)
            pltpu.sync_copy(dst_hbm.at[pl.ds(g * KC, KC)], dstv)

            def vec_body(j, cnt):
                d = dstv[pl.ds(j * 16, 16)]
                sv = srcv[pl.ds(j * 16, 16)]
                dl = d - lo
                m = (dl >= 0) & (dl < NB)
                key = jnp.where(m, 0, 1)
                _, pv = plsc.sort_key_val(key, sv * 1024 + dl)
                stage[pl.ds(cnt, 16)] = pv
                return cnt + plsc.all_reduce_population_count(m)[0]

            cnt = lax.fori_loop(0, KC // 16, vec_body, 0)
            stage[pl.ds(cnt, 16)] = dummy
            off = pl.multiple_of(w * CAP + cursor, 8)
            pltpu.sync_copy(stage.at[pl.ds(0, KC)],
                            epk_hbm.at[pl.ds(off, KC)])
            return cursor + ((cnt + 7) // 8) * 8

        cursor = lax.fori_loop(0, n_chunks, chunk_body, 0)
        def fill_body(t, _):
            stage[pl.ds(t * 16, 16)] = dummy
            return 0
        lax.fori_loop(0, KB // 16, fill_body, 0)
        off = pl.multiple_of(w * CAP + cursor, 8)
        pltpu.sync_copy(stage.at[pl.ds(0, KB)],
                        epk_hbm.at[pl.ds(off, KB)])
        total = ((cursor + KB - 1) // KB) * KB
        cstage[pl.ds(0, 16)] = jnp.full((16,), total, jnp.int32)
        pltpu.sync_copy(cstage, cnt_hbm.at[pl.ds(w * 16, 16)])

    return prep


@functools.lru_cache(maxsize=None)
def _make_segmin(NPAD, D, NB, CAP):
    mesh = plsc.VectorSubcoreMesh(core_axis_name="c", subcore_axis_name="s")

    @functools.partial(
        pl.kernel,
        out_type=jax.ShapeDtypeStruct((NPAD, D), jnp.float32),
        mesh=mesh,
        scratch_types=[pltpu.VMEM((NB + 1, D), jnp.float32),
                       pltpu.VMEM((KB,), jnp.int32),
                       pltpu.VMEM((KB,), jnp.int32),
                       pltpu.VMEM((KB, D), jnp.float32),
                       pltpu.VMEM((16,), jnp.int32),
                       pltpu.SemaphoreType.DMA],
    )
    def segmin(q_hbm, epk_hbm, cnt_hbm, out_hbm,
               acc, pbuf, eidx, rows, cv, sem):
        w = lax.axis_index("c") * 16 + lax.axis_index("s")
        pltpu.sync_copy(cnt_hbm.at[pl.ds(w * 16, 16)], cv)
        nblk = cv[pl.ds(0, 16)][0] // KB

        inf16 = jnp.full((16,), jnp.inf, jnp.float32)

        def init_body(i, _):
            for f in range(D // 16):
                acc[i, pl.ds(f * 16, 16)] = inf16
            return 0
        lax.fori_loop(0, NB + 1, init_body, 0)

        def block_body(g, _):
            off = pl.multiple_of(w * CAP + g * KB, 8)
            pltpu.sync_copy(epk_hbm.at[pl.ds(off, KB)], pbuf)

            def unpack_body(t, _):
                sl = pl.ds(t * 16, 16)
                eidx[sl] = lax.shift_right_logical(pbuf[sl], 10)
                return 0
            lax.fori_loop(0, KB // 16, unpack_body, 0)
            pltpu.async_copy(q_hbm.at[eidx], rows, sem).wait()

            def grp_body(j, _):
                dls = pbuf[pl.ds(j * 16, 16)] & 1023
                for k in range(16):
                    dl = dls[k]
                    e = j * 16 + k
                    for f in range(D // 16):
                        sl = pl.ds(f * 16, 16)
                        acc[dl, sl] = jnp.minimum(acc[dl, sl], rows[e, sl])
                return 0
            lax.fori_loop(0, KB // 16, grp_body, 0)
            return 0

        lax.fori_loop(0, nblk, block_body, 0)
        pltpu.sync_copy(acc.at[pl.ds(0, NB)], out_hbm.at[pl.ds(w * NB, NB)])

    return segmin


def _dense(body, outs, *args):
    return pl.pallas_call(body, out_shape=outs)(*args)


def kernel(x, edge_index, params):
    p = params
    n, d0 = x.shape
    e = edge_index.shape[1]
    nb = ((-(-n // NW)) + 7) // 8 * 8
    npad = nb * NW
    cap = e + KC + 2 * KB + 16
    cap = (cap + 7) // 8 * 8

    src, dst = edge_index[0], edge_index[1]
    epk, cnts = _make_edge_prep(e, nb, cap)(src, dst)

    xp = jnp.pad(x, ((0, npad - n), (0, 0)))
    w0 = p['first_W'] * (_INV * p['first_gamma'])[None, :]
    b0 = (p['first_b'] * _INV * p['first_gamma'] + p['first_beta'])[None, :]
    h, z = _dense(
        functools.partial(_first_body, n),
        (jax.ShapeDtypeStruct((npad, 128), jnp.float32),
         jax.ShapeDtypeStruct((1, 64), jnp.float32)),
        xp, w0, b0, p['lin0_W'], p['lin0_b'][None, :])
    out_feat = z

    for i in range(1, 4):
        s = _INV * p['conv%d_gamma' % i]
        wp = (p['conv%d_theta_W' % i] + p['conv%d_phi_W' % i]) * s[None, :]
        wq = p['conv%d_theta_W' % i] * s[None, :]
        c = ((p['conv%d_theta_b' % i] + p['conv%d_phi_b' % i]) * s
             + p['conv%d_beta' % i])[None, :]
        d = wp.shape[1]
        wq = jnp.pad(wq, ((0, 0), (0, 128 - d)))
        P, Q = _dense(
            _pq_body,
            (jax.ShapeDtypeStruct((npad, d), jnp.float32),
             jax.ShapeDtypeStruct((npad, 128), jnp.float32)),
            h, wp, wq)
        qmin = _make_segmin(npad, 128, nb, cap)(Q, epk, cnts)[:, :d]
        h, z = _dense(
            functools.partial(_combine_body, n),
            (jax.ShapeDtypeStruct((npad, d), jnp.float32),
             jax.ShapeDtypeStruct((1, 64), jnp.float32)),
            P, qmin, c, p['lin%d_W' % i], p['lin%d_b' % i][None, :])
        out_feat = out_feat + z

    out = out_feat @ p['cls_W'] + p['cls_b']
    return out, out_feat, h[:n]


# speedup vs baseline: 4.0563x; 1.2651x over previous
import functools
import numpy as np
import jax
import jax.numpy as jnp
from jax import lax
from jax.experimental import pallas as pl
from jax.experimental.pallas import tpu as pltpu, tpu_sc as plsc

_INV = float(1.0 / np.sqrt(1.0 + 1e-5))

NW = 32
KC = 4000
KB = 128


def _first_body(nvalid, x_ref, w_ref, b_ref, wl_ref, bl_ref, h_ref, z_ref):
    h = jnp.maximum(
        jnp.dot(x_ref[...], w_ref[...], preferred_element_type=jnp.float32)
        + b_ref[...], 0.0)
    rows = lax.broadcasted_iota(jnp.int32, h.shape, 0)
    h = jnp.where(rows < nvalid, h, 0.0)
    h_ref[...] = h
    cs = jnp.sum(h, axis=0, keepdims=True) * (1.0 / nvalid)
    z_ref[...] = jnp.dot(cs, wl_ref[...], preferred_element_type=jnp.float32) + bl_ref[...]


def _pq_body(h_ref, wp_ref, wq_ref, p_ref, q_ref):
    h = h_ref[...]
    p_ref[...] = jnp.dot(h, wp_ref[...], preferred_element_type=jnp.float32)
    q_ref[...] = jnp.dot(h, wq_ref[...], preferred_element_type=jnp.float32)


def _combine_body(nvalid, p_ref, qm_ref, c_ref, wl_ref, bl_ref, h_ref, z_ref):
    qm = qm_ref[...]
    agg = jnp.where(qm > 1e37, 0.0, p_ref[...] + c_ref[...] - qm)
    h = jnp.maximum(agg, 0.0)
    h_ref[...] = h
    cs = jnp.sum(h, axis=0, keepdims=True) * (1.0 / nvalid)
    z_ref[...] = jnp.dot(cs, wl_ref[...], preferred_element_type=jnp.float32) + bl_ref[...]


@functools.lru_cache(maxsize=None)
def _make_edge_prep(E, NB, CAP):
    mesh = plsc.VectorSubcoreMesh(core_axis_name="c", subcore_axis_name="s")
    n_chunks = E // KC
    assert E % KC == 0

    @functools.partial(
        pl.kernel,
        out_type=(jax.ShapeDtypeStruct((NW * CAP,), jnp.int32),
                  jax.ShapeDtypeStruct((NW * 16,), jnp.int32)),
        mesh=mesh,
        compiler_params=pltpu.CompilerParams(needs_layout_passes=False),
        scratch_types=[pltpu.VMEM((KC,), jnp.int32),
                       pltpu.VMEM((KC,), jnp.int32),
                       pltpu.VMEM((KC + 16,), jnp.int32),
                       pltpu.VMEM((16,), jnp.int32)],
    )
    def prep(src_hbm, dst_hbm, epk_hbm, cnt_hbm,
             srcv, dstv, stage, cstage):
        w = lax.axis_index("c") * 16 + lax.axis_index("s")
        lo = w * NB

        dummy = jnp.full((16,), NB, jnp.int32)

        def chunk_body(g, cursor):
            pltpu.sync_copy(src_hbm.at[pl.ds(g * KC, KC)], srcv)
            pltpu.sync_copy(dst_hbm.at[pl.ds(g * KC, KC)], dstv)

            def vec_body(j, cnt):
                pvs, pcs = [], []
                for u in range(5):
                    sl = pl.ds(j * 80 + u * 16, 16)
                    dl = dstv[sl] - lo
                    sv = srcv[sl]
                    m = (dl >= 0) & (dl < NB)
                    key = jnp.where(m, 0, 1)
                    _, pv = plsc.sort_key_val(key, sv * 1024 + dl)
                    pvs.append(pv)
                    pcs.append(plsc.all_reduce_population_count(m)[0])
                for u in range(5):
                    stage[pl.ds(cnt, 16)] = pvs[u]
                    cnt = cnt + pcs[u]
                return cnt

            cnt = lax.fori_loop(0, KC // 80, vec_body, 0)
            stage[pl.ds(cnt, 16)] = dummy
            off = pl.multiple_of(w * CAP + cursor, 8)
            pltpu.sync_copy(stage.at[pl.ds(0, KC)],
                            epk_hbm.at[pl.ds(off, KC)])
            return cursor + ((cnt + 7) // 8) * 8

        cursor = lax.fori_loop(0, n_chunks, chunk_body, 0)
        def fill_body(t, _):
            stage[pl.ds(t * 16, 16)] = dummy
            return 0
        lax.fori_loop(0, KB // 16, fill_body, 0)
        off = pl.multiple_of(w * CAP + cursor, 8)
        pltpu.sync_copy(stage.at[pl.ds(0, KB)],
                        epk_hbm.at[pl.ds(off, KB)])
        total = ((cursor + KB - 1) // KB) * KB
        cstage[pl.ds(0, 16)] = jnp.full((16,), total, jnp.int32)
        pltpu.sync_copy(cstage, cnt_hbm.at[pl.ds(w * 16, 16)])

    return prep


@functools.lru_cache(maxsize=None)
def _make_segmin(NPAD, D, NB, CAP):
    mesh = plsc.VectorSubcoreMesh(core_axis_name="c", subcore_axis_name="s")

    @functools.partial(
        pl.kernel,
        out_type=jax.ShapeDtypeStruct((NPAD, D), jnp.float32),
        mesh=mesh,
        scratch_types=[pltpu.VMEM((NB + 1, D), jnp.float32),
                       pltpu.VMEM((2, KB), jnp.int32),
                       pltpu.VMEM((2, KB), jnp.int32),
                       pltpu.VMEM((2, KB, D), jnp.float32),
                       pltpu.VMEM((16,), jnp.int32),
                       pltpu.SemaphoreType.DMA((2,))],
    )
    def segmin(q_hbm, epk_hbm, cnt_hbm, out_hbm,
               acc, pbuf, eidx, rows, cv, sems):
        w = lax.axis_index("c") * 16 + lax.axis_index("s")
        pltpu.sync_copy(cnt_hbm.at[pl.ds(w * 16, 16)], cv)
        nblk = cv[pl.ds(0, 16)][0] // KB

        inf16 = jnp.full((16,), jnp.inf, jnp.float32)

        def init_body(i, _):
            for f in range(D // 16):
                acc[i, pl.ds(f * 16, 16)] = inf16
            return 0
        lax.fori_loop(0, NB + 1, init_body, 0)

        def start(g, b):
            off = pl.multiple_of(w * CAP + g * KB, 8)
            pltpu.sync_copy(epk_hbm.at[pl.ds(off, KB)], pbuf.at[b])

            def unpack_body(t, _):
                sl = pl.ds(t * 16, 16)
                eidx[b, sl] = lax.shift_right_logical(pbuf[b, sl], 10)
                return 0
            lax.fori_loop(0, KB // 16, unpack_body, 0)
            pltpu.async_copy(q_hbm.at[eidx.at[b]], rows.at[b], sems.at[b])

        def wait(b):
            pltpu.make_async_copy(q_hbm.at[eidx.at[b]], rows.at[b],
                                  sems.at[b]).wait()

        @pl.when(nblk > 0)
        def _():
            start(0, 0)

        def block_body(g, _):
            b = lax.rem(g, 2)

            @pl.when(g + 1 < nblk)
            def _():
                start(g + 1, lax.rem(g + 1, 2))
            wait(b)

            def grp_body(j, _):
                dls = pbuf[b, pl.ds(j * 16, 16)] & 1023
                for k in range(16):
                    dl = dls[k]
                    e = j * 16 + k
                    rv = [rows[b, e, pl.ds(f * 16, 16)]
                          for f in range(D // 16)]
                    av = [acc[dl, pl.ds(f * 16, 16)]
                          for f in range(D // 16)]
                    for f in range(D // 16):
                        acc[dl, pl.ds(f * 16, 16)] = jnp.minimum(av[f], rv[f])
                return 0
            lax.fori_loop(0, KB // 16, grp_body, 0)
            return 0

        lax.fori_loop(0, nblk, block_body, 0)
        pltpu.sync_copy(acc.at[pl.ds(0, NB)], out_hbm.at[pl.ds(w * NB, NB)])

    return segmin


def _dense(body, outs, *args):
    return pl.pallas_call(body, out_shape=outs)(*args)


def kernel(x, edge_index, params):
    p = params
    n, d0 = x.shape
    e = edge_index.shape[1]
    nb = ((-(-n // NW)) + 7) // 8 * 8
    npad = nb * NW
    cap = e + KC + 2 * KB + 16
    cap = (cap + 7) // 8 * 8

    src, dst = edge_index[0], edge_index[1]
    epk, cnts = _make_edge_prep(e, nb, cap)(src, dst)

    xp = jnp.pad(x, ((0, npad - n), (0, 0)))
    w0 = p['first_W'] * (_INV * p['first_gamma'])[None, :]
    b0 = (p['first_b'] * _INV * p['first_gamma'] + p['first_beta'])[None, :]
    h, z = _dense(
        functools.partial(_first_body, n),
        (jax.ShapeDtypeStruct((npad, 128), jnp.float32),
         jax.ShapeDtypeStruct((1, 64), jnp.float32)),
        xp, w0, b0, p['lin0_W'], p['lin0_b'][None, :])
    out_feat = z

    for i in range(1, 4):
        s = _INV * p['conv%d_gamma' % i]
        wp = (p['conv%d_theta_W' % i] + p['conv%d_phi_W' % i]) * s[None, :]
        wq = p['conv%d_theta_W' % i] * s[None, :]
        c = ((p['conv%d_theta_b' % i] + p['conv%d_phi_b' % i]) * s
             + p['conv%d_beta' % i])[None, :]
        d = wp.shape[1]
        wq = jnp.pad(wq, ((0, 0), (0, 128 - d)))
        P, Q = _dense(
            _pq_body,
            (jax.ShapeDtypeStruct((npad, d), jnp.float32),
             jax.ShapeDtypeStruct((npad, 128), jnp.float32)),
            h, wp, wq)
        qmin = _make_segmin(npad, 128, nb, cap)(Q, epk, cnts)[:, :d]
        h, z = _dense(
            functools.partial(_combine_body, n),
            (jax.ShapeDtypeStruct((npad, d), jnp.float32),
             jax.ShapeDtypeStruct((1, 64), jnp.float32)),
            P, qmin, c, p['lin%d_W' % i], p['lin%d_b' % i][None, :])
        out_feat = out_feat + z

    out = out_feat @ p['cls_W'] + p['cls_b']
    return out, out_feat, h[:n]
